# baseline (speedup 1.0000x reference)
import functools

import jax
import jax.numpy as jnp
from jax import lax
from jax.experimental import pallas as pl
from jax.experimental.pallas import tpu as pltpu
from jax.experimental.pallas import tpu_sc as plsc

CK = 128
NB = 5
SPLITS = (120, 80)


def _prep_table(token_emb, V, D):
    tok_lin = jax.lax.optimization_barrier(jnp.reshape(token_emb, (V * D,)))
    return tok_lin.reshape(V, D)


def _make_gather(B, T, D, V, t0, ts):
    NC, NS = 2, 16
    NW = NC * NS
    ROWS = B * ts
    assert ROWS % (NW * 2 * CK) == 0 and D * 2 == 128
    M = ROWS // (NW * CK)
    assert M % NB == 0 and M >= 2 * NB

    mesh = plsc.VectorSubcoreMesh(core_axis_name="c", subcore_axis_name="s")

    @functools.partial(
        pl.kernel,
        mesh=mesh,
        out_type=jax.ShapeDtypeStruct((ROWS // 2, 2 * D), jnp.float32),
        scratch_types=[
            pltpu.VMEM((M * CK,), jnp.int32),
            pltpu.VMEM((CK, D), jnp.float32),
            pltpu.VMEM((CK, D), jnp.float32),
            pltpu.VMEM((CK, D), jnp.float32),
            pltpu.VMEM((CK, D), jnp.float32),
            pltpu.VMEM((CK, D), jnp.float32),
            pltpu.SemaphoreType.DMA,
            pltpu.SemaphoreType.DMA,
            pltpu.SemaphoreType.DMA,
            pltpu.SemaphoreType.DMA,
            pltpu.SemaphoreType.DMA,
            pltpu.SemaphoreType.DMA,
            pltpu.SemaphoreType.DMA,
            pltpu.SemaphoreType.DMA,
            pltpu.SemaphoreType.DMA,
            pltpu.SemaphoreType.DMA,
        ],
        compiler_params=pltpu.CompilerParams(use_tc_tiling_on_sc=False),
    )
    def k(x_hbm, tok_hbm, out_hbm,
          idx_all, b0, b1, b2, b3, b4,
          sg0, sg1, sg2, sg3, sg4, ss0, ss1, ss2, ss3, ss4):
        buf = (b0, b1, b2, b3, b4)
        sg = (sg0, sg1, sg2, sg3, sg4)
        ss = (ss0, ss1, ss2, ss3, ss4)

        wid = lax.axis_index("s") * NC + lax.axis_index("c")

        def gather(c, b):
            return pltpu.make_async_copy(
                tok_hbm.at[idx_all.at[pl.ds(c * CK, CK)]], buf[b], sg[b])

        def store(c, b):
            row0 = CK * ((M // 2) * wid) + CK * (c // 2)
            col = D * (c % 2)
            return pltpu.make_async_copy(
                buf[b],
                out_hbm.at[pl.ds(row0, CK), pl.ds(col, D)],
                ss[b])

        pltpu.sync_copy(
            x_hbm.at[pl.ds(t0 * B + wid * (M * CK), M * CK)], idx_all)
        gather(0, 0).start()
        gather(1, 1).start()
        gather(2, 2).start()

        def body(i, carry):
            for s in range(NB):
                c = NB * i + s
                b = s
                gather(c, b).wait()

                @pl.when(c >= 2)
                def _():
                    store(c - 2, (s - 2) % NB).wait()

                @pl.when(c + 3 < M)
                def _():
                    gather(c + 3, (s + 3) % NB).start()

                store(c, b).start()
            return carry

        lax.fori_loop(0, M // NB, body, 0)
        store(M - 2, (M - 2) % NB).wait()
        store(M - 1, (M - 1) % NB).wait()

    return k


def _make_finish(B, T, D, t0, ts, aliased):
    TB = 4
    RB = TB * B // 2
    NP = RB // CK
    T0 = t0
    TS = ts
    assert TS % TB == 0 and T0 % TB == 0

    if aliased:
        def body(o_in_ref, g_ref, pos_ref, o_ref):
            del o_in_ref
            _finish_block(g_ref, pos_ref, o_ref, TB, B, D, NP)
        in_specs = [
            pl.BlockSpec(memory_space=pl.ANY),
            pl.BlockSpec((RB, 2 * D), lambda i: (i, 0)),
            pl.BlockSpec((1, TB, D), lambda i: (T0 // TB + i, 0, 0)),
        ]
        io_aliases = {0: 0}
    else:
        def body(g_ref, pos_ref, o_ref):
            _finish_block(g_ref, pos_ref, o_ref, TB, B, D, NP)
        in_specs = [
            pl.BlockSpec((RB, 2 * D), lambda i: (i, 0)),
            pl.BlockSpec((1, TB, D), lambda i: (T0 // TB + i, 0, 0)),
        ]
        io_aliases = {}

    return pl.pallas_call(
        body,
        grid=(TS // TB,),
        in_specs=in_specs,
        out_specs=pl.BlockSpec((TB, D, B), lambda i: (T0 // TB + i, 0, 0)),
        out_shape=jax.ShapeDtypeStruct((T, D, B), jnp.float32),
        input_output_aliases=io_aliases,
    )


def _finish_block(g_ref, pos_ref, o_ref, TB, B, D, NP):
    v = g_ref[...]
    l3 = v[:, :D].reshape(NP, CK, D)
    r3 = v[:, D:].reshape(NP, CK, D)
    y = jnp.concatenate([l3, r3], axis=1)
    y = y.reshape(TB, B, D)
    z = jnp.swapaxes(y, 1, 2)
    o_ref[...] = z + pos_ref[0][:, :, None]


def kernel(x, token_emb, pos_emb):
    B, T = x.shape
    V, D = token_emb.shape
    xt_flat = jnp.transpose(x).reshape(T * B).astype(jnp.int32)
    pos3 = pos_emb.reshape(T // 4, 4, D)
    tok_rows = _prep_table(token_emb, V, D)
    assert sum(SPLITS) == T
    gs = []
    t0 = 0
    for ts in SPLITS:
        gs.append((t0, ts, _make_gather(B, T, D, V, t0, ts)(xt_flat, tok_rows)))
        t0 += ts
    o = None
    for t0, ts, g in gs:
        if o is None:
            o = _make_finish(B, T, D, t0, ts, aliased=False)(g, pos3)
        else:
            o = _make_finish(B, T, D, t0, ts, aliased=True)(o, g, pos3)
    return jnp.transpose(o, (2, 0, 1))


# speedup vs baseline: 1.0026x; 1.0026x over previous
import functools

import jax
import jax.numpy as jnp
from jax import lax
from jax.experimental import pallas as pl
from jax.experimental.pallas import tpu as pltpu
from jax.experimental.pallas import tpu_sc as plsc

CK = 128
NB = 5
SPLITS = (100, 60, 40)


def _prep_table(token_emb, V, D):
    tok_lin = jax.lax.optimization_barrier(jnp.reshape(token_emb, (V * D,)))
    return tok_lin.reshape(V, D)


def _make_gather(B, T, D, V, t0, ts):
    NC, NS = 2, 16
    NW = NC * NS
    ROWS = B * ts
    assert ROWS % (NW * 2 * CK) == 0 and D * 2 == 128
    M = ROWS // (NW * CK)
    assert M % NB == 0 and M >= 2 * NB

    mesh = plsc.VectorSubcoreMesh(core_axis_name="c", subcore_axis_name="s")

    @functools.partial(
        pl.kernel,
        mesh=mesh,
        out_type=jax.ShapeDtypeStruct((ROWS // 2, 2 * D), jnp.float32),
        scratch_types=[
            pltpu.VMEM((M * CK,), jnp.int32),
            pltpu.VMEM((CK, D), jnp.float32),
            pltpu.VMEM((CK, D), jnp.float32),
            pltpu.VMEM((CK, D), jnp.float32),
            pltpu.VMEM((CK, D), jnp.float32),
            pltpu.VMEM((CK, D), jnp.float32),
            pltpu.SemaphoreType.DMA,
            pltpu.SemaphoreType.DMA,
            pltpu.SemaphoreType.DMA,
            pltpu.SemaphoreType.DMA,
            pltpu.SemaphoreType.DMA,
            pltpu.SemaphoreType.DMA,
            pltpu.SemaphoreType.DMA,
            pltpu.SemaphoreType.DMA,
            pltpu.SemaphoreType.DMA,
            pltpu.SemaphoreType.DMA,
        ],
        compiler_params=pltpu.CompilerParams(use_tc_tiling_on_sc=False),
    )
    def k(x_hbm, tok_hbm, out_hbm,
          idx_all, b0, b1, b2, b3, b4,
          sg0, sg1, sg2, sg3, sg4, ss0, ss1, ss2, ss3, ss4):
        buf = (b0, b1, b2, b3, b4)
        sg = (sg0, sg1, sg2, sg3, sg4)
        ss = (ss0, ss1, ss2, ss3, ss4)

        wid = lax.axis_index("s") * NC + lax.axis_index("c")

        def gather(c, b):
            return pltpu.make_async_copy(
                tok_hbm.at[idx_all.at[pl.ds(c * CK, CK)]], buf[b], sg[b])

        def store(c, b):
            row0 = CK * ((M // 2) * wid) + CK * (c // 2)
            col = D * (c % 2)
            return pltpu.make_async_copy(
                buf[b],
                out_hbm.at[pl.ds(row0, CK), pl.ds(col, D)],
                ss[b])

        pltpu.sync_copy(
            x_hbm.at[pl.ds(t0 * B + wid * (M * CK), M * CK)], idx_all)
        gather(0, 0).start()
        gather(1, 1).start()
        gather(2, 2).start()

        def body(i, carry):
            for s in range(NB):
                c = NB * i + s
                b = s
                gather(c, b).wait()

                @pl.when(c >= 2)
                def _():
                    store(c - 2, (s - 2) % NB).wait()

                @pl.when(c + 3 < M)
                def _():
                    gather(c + 3, (s + 3) % NB).start()

                store(c, b).start()
            return carry

        lax.fori_loop(0, M // NB, body, 0)
        store(M - 2, (M - 2) % NB).wait()
        store(M - 1, (M - 1) % NB).wait()

    return k


def _make_finish(B, T, D, t0, ts, aliased):
    TB = 4
    RB = TB * B // 2
    NP = RB // CK
    T0 = t0
    TS = ts
    assert TS % TB == 0 and T0 % TB == 0

    if aliased:
        def body(o_in_ref, g_ref, pos_ref, o_ref):
            del o_in_ref
            _finish_block(g_ref, pos_ref, o_ref, TB, B, D, NP)
        in_specs = [
            pl.BlockSpec(memory_space=pl.ANY),
            pl.BlockSpec((RB, 2 * D), lambda i: (i, 0)),
            pl.BlockSpec((1, TB, D), lambda i: (T0 // TB + i, 0, 0)),
        ]
        io_aliases = {0: 0}
    else:
        def body(g_ref, pos_ref, o_ref):
            _finish_block(g_ref, pos_ref, o_ref, TB, B, D, NP)
        in_specs = [
            pl.BlockSpec((RB, 2 * D), lambda i: (i, 0)),
            pl.BlockSpec((1, TB, D), lambda i: (T0 // TB + i, 0, 0)),
        ]
        io_aliases = {}

    return pl.pallas_call(
        body,
        grid=(TS // TB,),
        in_specs=in_specs,
        out_specs=pl.BlockSpec((TB, D, B), lambda i: (T0 // TB + i, 0, 0)),
        out_shape=jax.ShapeDtypeStruct((T, D, B), jnp.float32),
        input_output_aliases=io_aliases,
    )


def _finish_block(g_ref, pos_ref, o_ref, TB, B, D, NP):
    v = g_ref[...]
    l3 = v[:, :D].reshape(NP, CK, D)
    r3 = v[:, D:].reshape(NP, CK, D)
    y = jnp.concatenate([l3, r3], axis=1)
    y = y.reshape(TB, B, D)
    z = jnp.swapaxes(y, 1, 2)
    o_ref[...] = z + pos_ref[0][:, :, None]


def kernel(x, token_emb, pos_emb):
    B, T = x.shape
    V, D = token_emb.shape
    xt_flat = jnp.transpose(x).reshape(T * B).astype(jnp.int32)
    pos3 = pos_emb.reshape(T // 4, 4, D)
    tok_rows = _prep_table(token_emb, V, D)
    assert sum(SPLITS) == T
    gs = []
    t0 = 0
    for ts in SPLITS:
        gs.append((t0, ts, _make_gather(B, T, D, V, t0, ts)(xt_flat, tok_rows)))
        t0 += ts
    o = None
    for t0, ts, g in gs:
        if o is None:
            o = _make_finish(B, T, D, t0, ts, aliased=False)(g, pos3)
        else:
            o = _make_finish(B, T, D, t0, ts, aliased=True)(o, g, pos3)
    return jnp.transpose(o, (2, 0, 1))


# speedup vs baseline: 1.0392x; 1.0365x over previous
import functools

import jax
import jax.numpy as jnp
from jax import lax
from jax.experimental import pallas as pl
from jax.experimental.pallas import tpu as pltpu
from jax.experimental.pallas import tpu_sc as plsc

CK = 128
NB = 5
SPLITS = (40, 60, 100)
PBK = 1024


def _make_pack(V, D):
    VP = ((V + 2 * PBK - 1) // (2 * PBK)) * 2 * PBK
    grid = VP // (2 * PBK)

    def body(a_ref, b_ref, o_ref):
        wa = jnp.swapaxes(a_ref[...], 0, 1)
        wb = jnp.swapaxes(b_ref[...], 0, 1)
        o_ref[...] = jnp.concatenate([wa, wb], axis=1)

    return VP, pl.pallas_call(
        body,
        grid=(grid,),
        in_specs=[
            pl.BlockSpec((D, PBK), lambda i: (0, 2 * i)),
            pl.BlockSpec((D, PBK), lambda i: (0, 2 * i + 1)),
        ],
        out_specs=pl.BlockSpec((PBK, 2 * D), lambda i: (i, 0)),
        out_shape=jax.ShapeDtypeStruct((VP // 2, 2 * D), jnp.float32),
    )


def _permute_idx(v):
    blk = v // (2 * PBK)
    jj = v % (2 * PBK)
    return 2 * (PBK * blk + (jj % PBK)) + jj // PBK


def _make_gather(B, T, D, V, t0, ts):
    NC, NS = 2, 16
    NW = NC * NS
    ROWS = B * ts
    assert ROWS % (NW * 2 * CK) == 0 and D * 2 == 128
    M = ROWS // (NW * CK)
    assert M % NB == 0 and M >= 2 * NB

    mesh = plsc.VectorSubcoreMesh(core_axis_name="c", subcore_axis_name="s")

    @functools.partial(
        pl.kernel,
        mesh=mesh,
        out_type=jax.ShapeDtypeStruct((ROWS // 2, 2 * D), jnp.float32),
        scratch_types=[
            pltpu.VMEM((M * CK,), jnp.int32),
            pltpu.VMEM((CK, D), jnp.float32),
            pltpu.VMEM((CK, D), jnp.float32),
            pltpu.VMEM((CK, D), jnp.float32),
            pltpu.VMEM((CK, D), jnp.float32),
            pltpu.VMEM((CK, D), jnp.float32),
            pltpu.SemaphoreType.DMA,
            pltpu.SemaphoreType.DMA,
            pltpu.SemaphoreType.DMA,
            pltpu.SemaphoreType.DMA,
            pltpu.SemaphoreType.DMA,
            pltpu.SemaphoreType.DMA,
            pltpu.SemaphoreType.DMA,
            pltpu.SemaphoreType.DMA,
            pltpu.SemaphoreType.DMA,
            pltpu.SemaphoreType.DMA,
        ],
        compiler_params=pltpu.CompilerParams(use_tc_tiling_on_sc=False),
    )
    def k(x_hbm, tok_hbm, out_hbm,
          idx_all, b0, b1, b2, b3, b4,
          sg0, sg1, sg2, sg3, sg4, ss0, ss1, ss2, ss3, ss4):
        buf = (b0, b1, b2, b3, b4)
        sg = (sg0, sg1, sg2, sg3, sg4)
        ss = (ss0, ss1, ss2, ss3, ss4)

        wid = lax.axis_index("s") * NC + lax.axis_index("c")

        def gather(c, b):
            return pltpu.make_async_copy(
                tok_hbm.at[idx_all.at[pl.ds(c * CK, CK)]], buf[b], sg[b])

        def store(c, b):
            row0 = CK * ((M // 2) * wid) + CK * (c // 2)
            col = D * (c % 2)
            return pltpu.make_async_copy(
                buf[b],
                out_hbm.at[pl.ds(row0, CK), pl.ds(col, D)],
                ss[b])

        pltpu.sync_copy(
            x_hbm.at[pl.ds(t0 * B + wid * (M * CK), M * CK)], idx_all)
        gather(0, 0).start()
        gather(1, 1).start()
        gather(2, 2).start()

        def body(i, carry):
            for s in range(NB):
                c = NB * i + s
                b = s
                gather(c, b).wait()

                @pl.when(c >= 2)
                def _():
                    store(c - 2, (s - 2) % NB).wait()

                @pl.when(c + 3 < M)
                def _():
                    gather(c + 3, (s + 3) % NB).start()

                store(c, b).start()
            return carry

        lax.fori_loop(0, M // NB, body, 0)
        store(M - 2, (M - 2) % NB).wait()
        store(M - 1, (M - 1) % NB).wait()

    return k


def _make_finish(B, T, D, t0, ts, aliased):
    TB = 4
    RB = TB * B // 2
    NP = RB // CK
    T0 = t0
    TS = ts
    assert TS % TB == 0 and T0 % TB == 0

    if aliased:
        def body(o_in_ref, g_ref, pos_ref, o_ref):
            del o_in_ref
            _finish_block(g_ref, pos_ref, o_ref, TB, B, D, NP)
        in_specs = [
            pl.BlockSpec(memory_space=pl.ANY),
            pl.BlockSpec((RB, 2 * D), lambda i: (i, 0)),
            pl.BlockSpec((1, TB, D), lambda i: (T0 // TB + i, 0, 0)),
        ]
        io_aliases = {0: 0}
    else:
        def body(g_ref, pos_ref, o_ref):
            _finish_block(g_ref, pos_ref, o_ref, TB, B, D, NP)
        in_specs = [
            pl.BlockSpec((RB, 2 * D), lambda i: (i, 0)),
            pl.BlockSpec((1, TB, D), lambda i: (T0 // TB + i, 0, 0)),
        ]
        io_aliases = {}

    return pl.pallas_call(
        body,
        grid=(TS // TB,),
        in_specs=in_specs,
        out_specs=pl.BlockSpec((TB, D, B), lambda i: (T0 // TB + i, 0, 0)),
        out_shape=jax.ShapeDtypeStruct((T, D, B), jnp.float32),
        input_output_aliases=io_aliases,
    )


def _finish_block(g_ref, pos_ref, o_ref, TB, B, D, NP):
    v = g_ref[...]
    l3 = v[:, :D].reshape(NP, CK, D)
    r3 = v[:, D:].reshape(NP, CK, D)
    y = jnp.concatenate([l3, r3], axis=1)
    y = y.reshape(TB, B, D)
    z = jnp.swapaxes(y, 1, 2)
    o_ref[...] = z + pos_ref[0][:, :, None]


def kernel(x, token_emb, pos_emb):
    B, T = x.shape
    V, D = token_emb.shape
    xt = jnp.transpose(x).reshape(T * B).astype(jnp.int32)
    xt_flat = _permute_idx(xt).astype(jnp.int32)
    pos3 = pos_emb.reshape(T // 4, 4, D)
    VP, pack = _make_pack(V, D)
    tokT = jnp.transpose(token_emb)
    tok_rows = pack(tokT, tokT).reshape(VP, D)
    assert sum(SPLITS) == T
    gs = []
    t0 = 0
    for ts in SPLITS:
        gs.append((t0, ts, _make_gather(B, T, D, V, t0, ts)(xt_flat, tok_rows)))
        t0 += ts
    o = None
    for t0, ts, g in gs:
        if o is None:
            o = _make_finish(B, T, D, t0, ts, aliased=False)(g, pos3)
        else:
            o = _make_finish(B, T, D, t0, ts, aliased=True)(o, g, pos3)
    return jnp.transpose(o, (2, 0, 1))
